# Initial kernel scaffold; baseline (speedup 1.0000x reference)
import functools

import jax
import jax.numpy as jnp
from jax import lax
from jax.experimental import pallas as pl
from jax.experimental.pallas import tpu as pltpu
from jax.experimental.pallas import tpu_sc as plsc

_D = 128
_NC = 2
_NS = 16
_NW = _NC * _NS
_CHUNK = 128


@functools.lru_cache(maxsize=None)
def _build(n_idx):
    assert n_idx % (_NW * _CHUNK) == 0
    n_chunks_w = n_idx // (_NW * _CHUNK)
    b_per_w = n_idx // _NW

    @functools.partial(
        pl.kernel,
        out_type=jax.ShapeDtypeStruct((n_idx, _D), jnp.float32),
        mesh=plsc.VectorSubcoreMesh(core_axis_name="c", subcore_axis_name="s"),
        scratch_types=[
            pltpu.VMEM((n_chunks_w, _CHUNK), jnp.int32),
            pltpu.VMEM((_CHUNK, _D), jnp.float32),
            pltpu.SemaphoreType.DMA,
        ],
    )
    def emb(table_hbm, idx_hbm, out_hbm, idx_v, rows_v, sem):
        wid = lax.axis_index("s") * _NC + lax.axis_index("c")
        base = wid * b_per_w
        pltpu.sync_copy(idx_hbm.at[pl.ds(wid * n_chunks_w, n_chunks_w)], idx_v)

        def body(j, carry):
            pltpu.async_copy(table_hbm.at[idx_v.at[j]], rows_v, sem).wait()
            pltpu.sync_copy(rows_v, out_hbm.at[pl.ds(base + j * _CHUNK, _CHUNK)])
            return carry

        lax.fori_loop(0, n_chunks_w, body, 0)

    return emb


def kernel(inputs, embedding_table):
    b, s = inputs.shape
    n = b * s
    idx2d = inputs.reshape(n // _CHUNK, _CHUNK).astype(jnp.int32)
    out = _build(n)(embedding_table, idx2d)
    return out.reshape(b, s, _D), embedding_table


# speedup vs baseline: 2.7867x; 2.7867x over previous
import functools

import jax
import jax.numpy as jnp
from jax import lax
from jax.experimental import pallas as pl
from jax.experimental.pallas import tpu as pltpu
from jax.experimental.pallas import tpu_sc as plsc

_D = 128
_NC = 2
_NS = 16
_NW = _NC * _NS
_CHUNK = 128


@functools.lru_cache(maxsize=None)
def _build(n_idx):
    assert n_idx % (_NW * _CHUNK) == 0
    n_chunks_w = n_idx // (_NW * _CHUNK)
    b_per_w = n_idx // _NW

    @functools.partial(
        pl.kernel,
        out_type=jax.ShapeDtypeStruct((n_idx, _D), jnp.float32),
        mesh=plsc.VectorSubcoreMesh(core_axis_name="c", subcore_axis_name="s"),
        scratch_types=[
            pltpu.VMEM((b_per_w,), jnp.int32),
            pltpu.VMEM((_CHUNK, _D), jnp.float32),
            pltpu.SemaphoreType.DMA,
        ],
    )
    def emb(table_hbm, idx_hbm, out_hbm, idx_v, rows_v, sem):
        wid = lax.axis_index("s") * _NC + lax.axis_index("c")
        base = wid * b_per_w
        pltpu.sync_copy(idx_hbm.at[pl.ds(base, b_per_w)], idx_v)

        def body(j, carry):
            pltpu.async_copy(
                table_hbm.at[idx_v.at[pl.ds(j * _CHUNK, _CHUNK)]], rows_v, sem
            ).wait()
            pltpu.sync_copy(rows_v, out_hbm.at[pl.ds(base + j * _CHUNK, _CHUNK)])
            return carry

        lax.fori_loop(0, n_chunks_w, body, 0)

    return emb


def kernel(inputs, embedding_table):
    b, s = inputs.shape
    n = b * s
    idx = inputs.reshape(n).astype(jnp.int32)
    out = _build(n)(embedding_table, idx)
    return out.reshape(b, s, _D), embedding_table


# speedup vs baseline: 3.1051x; 1.1143x over previous
import functools

import jax
import jax.numpy as jnp
from jax import lax
from jax.experimental import pallas as pl
from jax.experimental.pallas import tpu as pltpu
from jax.experimental.pallas import tpu_sc as plsc

_D = 128
_NC = 2
_NS = 16
_NW = _NC * _NS
_CHUNK = 128
_NB = 5
_LA = 2


@functools.lru_cache(maxsize=None)
def _build(n_idx):
    assert n_idx % (_NW * _CHUNK) == 0
    n_chunks = n_idx // (_NW * _CHUNK)
    b_per_w = n_idx // _NW
    assert n_chunks % _NB == 0 and n_chunks // _NB >= 2
    n_groups = n_chunks // _NB

    @functools.partial(
        pl.kernel,
        out_type=jax.ShapeDtypeStruct((n_idx, _D), jnp.float32),
        mesh=plsc.VectorSubcoreMesh(core_axis_name="c", subcore_axis_name="s"),
        scratch_types=[
            pltpu.VMEM((b_per_w,), jnp.int32),
            [pltpu.VMEM((_CHUNK, _D), jnp.float32) for _ in range(_NB)],
            [pltpu.SemaphoreType.DMA for _ in range(_NB)],
            [pltpu.SemaphoreType.DMA for _ in range(_NB)],
        ],
    )
    def emb(table_hbm, idx_hbm, out_hbm, idx_v, bufs, gsems, osems):
        wid = lax.axis_index("s") * _NC + lax.axis_index("c")
        base = wid * b_per_w
        pltpu.sync_copy(idx_hbm.at[pl.ds(base, b_per_w)], idx_v)

        def start_gather(j, b):
            pltpu.async_copy(
                table_hbm.at[idx_v.at[pl.ds(j * _CHUNK, _CHUNK)]],
                bufs[b], gsems[b],
            )

        def wait_gather(b):
            pltpu.make_async_copy(table_hbm.at[pl.ds(0, _CHUNK)], bufs[b],
                                  gsems[b]).wait()

        def start_out(j, b):
            pltpu.async_copy(bufs[b], out_hbm.at[pl.ds(base + j * _CHUNK, _CHUNK)],
                             osems[b])

        def wait_out(b):
            pltpu.make_async_copy(bufs[b], out_hbm.at[pl.ds(0, _CHUNK)],
                                  osems[b]).wait()

        for b in range(_LA):
            start_gather(b, b)

        for b in range(_NB):
            jn = b + _LA
            if jn < n_chunks:
                bb = jn % _NB
                if jn >= _NB:
                    wait_out(bb)
                start_gather(jn, bb)
            wait_gather(b)
            start_out(b, b)

        def group(g, carry):
            j0 = g * _NB
            for b in range(_NB):
                j = j0 + b
                bb = (b + _LA) % _NB
                wait_out(bb)
                start_gather(j + _LA, bb)
                wait_gather(b)
                start_out(j, b)
            return carry

        lax.fori_loop(1, n_groups - 1, group, 0)

        j0 = (n_groups - 1) * _NB
        for b in range(_NB):
            j = j0 + b
            jn = j + _LA
            if jn < n_chunks:
                bb = jn % _NB
                wait_out(bb)
                start_gather(jn, bb)
            wait_gather(b)
            start_out(j, b)

        for b in range(_NB):
            wait_out(b)

    return emb


def kernel(inputs, embedding_table):
    b, s = inputs.shape
    n = b * s
    idx = inputs.reshape(n).astype(jnp.int32)
    out = _build(n)(embedding_table, idx)
    return out.reshape(b, s, _D), embedding_table


# speedup vs baseline: 5.1296x; 1.6520x over previous
import functools

import jax
import jax.numpy as jnp
from jax import lax
from jax.experimental import pallas as pl
from jax.experimental.pallas import tpu as pltpu
from jax.experimental.pallas import tpu_sc as plsc

_D = 128
_NC = 2
_NS = 16
_NW = _NC * _NS
_NB = 8
_LA = 3


@functools.lru_cache(maxsize=None)
def _build(batch, seq):
    assert batch % (_NW * _NB) == 0
    n_chunks = batch // _NW
    n_groups = n_chunks // _NB
    assert n_groups >= 2

    @functools.partial(
        pl.kernel,
        out_type=jax.ShapeDtypeStruct((batch, seq, _D), jnp.float32),
        mesh=plsc.VectorSubcoreMesh(core_axis_name="c", subcore_axis_name="s"),
        scratch_types=[
            pltpu.VMEM((n_chunks, seq), jnp.int32),
            [pltpu.VMEM((seq, _D), jnp.float32) for _ in range(_NB)],
            [pltpu.SemaphoreType.DMA for _ in range(_NB)],
            [pltpu.SemaphoreType.DMA for _ in range(_NB)],
        ],
    )
    def emb(table_hbm, idx_hbm, out_hbm, idx_v, bufs, gsems, osems):
        wid = lax.axis_index("s") * _NC + lax.axis_index("c")
        base = wid * n_chunks
        pltpu.sync_copy(idx_hbm.at[pl.ds(base, n_chunks)], idx_v)

        def start_gather(j, b):
            pltpu.async_copy(table_hbm.at[idx_v.at[j]], bufs[b], gsems[b])

        def wait_gather(j, b):
            pltpu.make_async_copy(table_hbm.at[idx_v.at[j]], bufs[b],
                                  gsems[b]).wait()

        def start_out(j, b):
            pltpu.async_copy(bufs[b], out_hbm.at[base + j], osems[b])

        def wait_out(b):
            pltpu.make_async_copy(bufs[b], out_hbm.at[0], osems[b]).wait()

        for b in range(_LA):
            start_gather(b, b)

        for b in range(_NB):
            jn = b + _LA
            if jn < n_chunks:
                bb = jn % _NB
                if jn >= _NB:
                    wait_out(bb)
                start_gather(jn, bb)
            wait_gather(b, b)
            start_out(b, b)

        def group(g, carry):
            j0 = g * _NB
            for b in range(_NB):
                j = j0 + b
                bb = (b + _LA) % _NB
                wait_out(bb)
                start_gather(j + _LA, bb)
                wait_gather(j, b)
                start_out(j, b)
            return carry

        lax.fori_loop(1, n_groups - 1, group, 0)

        j0 = (n_groups - 1) * _NB
        for b in range(_NB):
            j = j0 + b
            jn = j + _LA
            if jn < n_chunks:
                bb = jn % _NB
                wait_out(bb)
                start_gather(jn, bb)
            wait_gather(j, b)
            start_out(j, b)

        for b in range(_NB):
            wait_out(b)

    return emb


def kernel(inputs, embedding_table):
    b, s = inputs.shape
    out = _build(b, s)(embedding_table, inputs.astype(jnp.int32))
    return out, embedding_table
